# baseline (speedup 1.0000x reference)
import functools

import jax
import jax.numpy as jnp
from jax import lax
from jax.experimental import pallas as pl
from jax.experimental.pallas import tpu as pltpu
from jax.experimental.pallas import tpu_sc as plsc

N = 10000
E = 320000
F = 128
NC = 2
NS = 16
NW = NC * NS
CHUNK = 96
NCHUNK = 108
EP = NW * NCHUNK * CHUNK
NPAD = 10240
BR = 1000
BRP = 1024


def _sc_deg(col3, ew3, zeros_n):
    mesh = plsc.VectorSubcoreMesh(core_axis_name="c", subcore_axis_name="s")

    @functools.partial(
        pl.kernel,
        mesh=mesh,
        out_type=jax.ShapeDtypeStruct((NC, NPAD), jnp.float32),
        scratch_types=[
            pltpu.VMEM((NCHUNK, CHUNK), jnp.int32),
            pltpu.VMEM((NCHUNK, CHUNK), jnp.float32),
            pltpu.VMEM_SHARED((NPAD,), jnp.float32),
            pltpu.SemaphoreType.DMA,
        ],
    )
    def k(colh, ewh, zh, degout, colv, eww, deg, sem):
        c = lax.axis_index("c")
        s = lax.axis_index("s")
        w = c * NS + s
        sl = pl.ds(s * (NPAD // NS), NPAD // NS)
        pltpu.sync_copy(zh.at[sl], deg.at[sl])
        plsc.subcore_barrier()
        pltpu.sync_copy(colh.at[w], colv)
        pltpu.sync_copy(ewh.at[w], eww)

        def group(g, carry):
            base = g * 6
            for t in range(6):
                pltpu.async_copy(eww.at[base + t],
                                 deg.at[colv.at[base + t]], sem, add=True)
            for t in range(6):
                pltpu.make_async_copy(eww.at[base + t],
                                      deg.at[colv.at[base + t]], sem).wait()
            return carry

        lax.fori_loop(0, NCHUNK // 6, group, 0)
        plsc.subcore_barrier()
        pltpu.sync_copy(deg.at[sl], degout.at[c, sl])

    return k(col3, ew3, zeros_n)


def _xs_body(x_ref, d0_ref, d1_ref, o_ref):
    ds = lax.rsqrt(d0_ref[...] + d1_ref[...] + 1.0)
    o_ref[...] = x_ref[...] * ds


def _tc_xs(x2, d0, d1):
    return pl.pallas_call(
        _xs_body,
        grid=(NPAD // BRP,),
        in_specs=[
            pl.BlockSpec((BRP, F), lambda i: (i, 0)),
            pl.BlockSpec((BRP, 1), lambda i: (i, 0)),
            pl.BlockSpec((BRP, 1), lambda i: (i, 0)),
        ],
        out_specs=pl.BlockSpec((BRP, F), lambda i: (i, 0)),
        out_shape=jax.ShapeDtypeStruct((NPAD, F), jnp.float32),
    )(x2, d0, d1)


def _sc_agg(row3, col3, ew3, xs, zeros_nf):
    mesh = plsc.VectorSubcoreMesh(core_axis_name="c", subcore_axis_name="s")

    @functools.partial(
        pl.kernel,
        mesh=mesh,
        out_type=jax.ShapeDtypeStruct((NC, NPAD, F), jnp.float32),
        scratch_types=[
            pltpu.VMEM((6, CHUNK), jnp.int32),
            pltpu.VMEM((6, CHUNK), jnp.int32),
            pltpu.VMEM((6, CHUNK), jnp.float32),
            pltpu.VMEM((CHUNK, F), jnp.float32),
            pltpu.VMEM((CHUNK, F), jnp.float32),
            pltpu.VMEM((CHUNK, F), jnp.float32),
            pltpu.SemaphoreType.DMA,
            pltpu.SemaphoreType.DMA,
            pltpu.SemaphoreType.DMA,
            pltpu.SemaphoreType.DMA,
            pltpu.SemaphoreType.DMA,
            pltpu.SemaphoreType.DMA,
            pltpu.SemaphoreType.DMA,
            pltpu.SemaphoreType.DMA,
            pltpu.SemaphoreType.DMA,
            pltpu.SemaphoreType.DMA,
            pltpu.SemaphoreType.DMA,
            pltpu.SemaphoreType.DMA,
            pltpu.VMEM_SHARED((NPAD, F), jnp.float32),
        ],
    )
    def k(rowh, colh, ewh, xsh, zh, aggout, rowi, coli, ewc,
          rows0, rows1, rows2,
          is0, is1, is2, is3, is4, is5, gs0, gs1, gs2, ss0, ss1, ss2, agg):
        c = lax.axis_index("c")
        s = lax.axis_index("s")
        w = c * NS + s
        sl = pl.ds(s * (NPAD // NS), NPAD // NS)

        @pl.when(c == 0)
        def _():
            pltpu.sync_copy(xsh.at[sl], agg.at[sl])

        @pl.when(c != 0)
        def _():
            pltpu.sync_copy(zh.at[sl], agg.at[sl])

        plsc.subcore_barrier()

        bufs = (rows0, rows1, rows2)
        isems = (is0, is1, is2, is3, is4, is5)
        gsems = (gs0, gs1, gs2)
        ssems = (ss0, ss1, ss2)
        dnums = lax.GatherDimensionNumbers(
            offset_dims=(), collapsed_slice_dims=(0,), start_index_map=(0,))

        def idx_start(cix, q):
            pltpu.async_copy(rowh.at[w, cix], rowi.at[q], isems[q])
            pltpu.async_copy(colh.at[w, cix], coli.at[q], isems[q])
            pltpu.async_copy(ewh.at[w, cix], ewc.at[q], isems[q])

        def idx_wait(cix, q):
            pltpu.make_async_copy(rowh.at[w, cix], rowi.at[q], isems[q]).wait()
            pltpu.make_async_copy(colh.at[w, cix], coli.at[q], isems[q]).wait()
            pltpu.make_async_copy(ewh.at[w, cix], ewc.at[q], isems[q]).wait()

        def gather_start(q, b):
            pltpu.async_copy(xsh.at[rowi.at[q]], bufs[b], gsems[b])

        def gather_wait(q, b):
            pltpu.make_async_copy(xsh.at[rowi.at[q]], bufs[b],
                                  gsems[b]).wait()

        def scatter_start(q, b):
            pltpu.async_copy(bufs[b], agg.at[coli.at[q]], ssems[b],
                             add=True)

        def scatter_wait(q, b):
            pltpu.make_async_copy(bufs[b], agg.at[coli.at[q]],
                                  ssems[b]).wait()

        def scale(q, b):
            rows = bufs[b]

            def grp(g, c2):
                wgrp = ewc[q, pl.ds(g * 16, 16)]
                for l in range(16):
                    wv = lax.gather(
                        wgrp, jnp.full((16, 1), l, jnp.int32), dnums, (1,),
                        mode=lax.GatherScatterMode.PROMISE_IN_BOUNDS)
                    kk = g * 16 + l
                    for t in range(F // 16):
                        rows[kk, pl.ds(t * 16, 16)] = (
                            rows[kk, pl.ds(t * 16, 16)] * wv)
                return c2

            lax.fori_loop(0, CHUNK // 16, grp, 0)

        for c0 in range(4):
            idx_start(c0, c0)
        idx_wait(0, 0)
        gather_start(0, 0)
        for c0 in (0, 1):
            gather_wait(c0 % 6, c0 % 3)
            idx_wait(c0 + 1, c0 + 1)
            gather_start((c0 + 1) % 6, (c0 + 1) % 3)
            idx_start(c0 + 4, c0 + 4)
            scale(c0 % 6, c0 % 3)
            scatter_start(c0 % 6, c0 % 3)

        def steady(i, carry):
            j = 2 + i * 6
            for p in range(6):
                cix = j + p
                q = (2 + p) % 6
                b = (2 + p) % 3
                qn = (3 + p) % 6
                bn = (p + 0) % 3
                gather_wait(q, b)
                scatter_wait((p + 0) % 6, bn)
                idx_wait(cix + 1, qn)
                gather_start(qn, bn)
                idx_start(cix + 4, (p + 0) % 6)
                scale(q, b)
                scatter_start(q, b)
            return carry

        lax.fori_loop(0, (NCHUNK - 6) // 6, steady, 0)

        for c0 in (NCHUNK - 4, NCHUNK - 3, NCHUNK - 2):
            gather_wait(c0 % 6, c0 % 3)
            scatter_wait((c0 - 2) % 6, (c0 + 1) % 3)
            idx_wait(c0 + 1, (c0 + 1) % 6)
            gather_start((c0 + 1) % 6, (c0 + 1) % 3)
            scale(c0 % 6, c0 % 3)
            scatter_start(c0 % 6, c0 % 3)
        c0 = NCHUNK - 1
        gather_wait(c0 % 6, c0 % 3)
        scatter_wait((c0 - 2) % 6, (c0 + 1) % 3)
        scale(c0 % 6, c0 % 3)
        scatter_start(c0 % 6, c0 % 3)
        scatter_wait((NCHUNK - 2) % 6, (NCHUNK - 2) % 3)
        scatter_wait((NCHUNK - 1) % 6, (NCHUNK - 1) % 3)

        plsc.subcore_barrier()
        pltpu.sync_copy(agg.at[sl], aggout.at[c, sl])

    return k(row3, col3, ew3, xs, zeros_nf)


def _gru_body(a0_ref, a1_ref, h_ref, d0_ref, d1_ref,
              wz_ref, wr_ref, wh_ref, lz_ref, lr_ref, lh_ref,
              bz_ref, br_ref, bh_ref, o_ref):
    f32 = jnp.float32
    ds = lax.rsqrt(d0_ref[...] + d1_ref[...] + 1.0)
    agg = (a0_ref[...] + a1_ref[...]) * ds
    h = h_ref[...]

    def gate(w_ref, l_ref, b_ref, hv):
        lt = l_ref[0:F, :]
        lb = l_ref[F:2 * F, :]
        a = jnp.dot(w_ref[...], lt, preferred_element_type=f32)
        pre = (jnp.dot(agg, a, preferred_element_type=f32)
               + jnp.dot(hv, lb, preferred_element_type=f32)
               + b_ref[...])
        return pre

    z = jax.nn.sigmoid(gate(wz_ref, lz_ref, bz_ref, h))
    r = jax.nn.sigmoid(gate(wr_ref, lr_ref, br_ref, h))
    ht = jnp.tanh(gate(wh_ref, lh_ref, bh_ref, h * r))
    o_ref[...] = z * h + (1.0 - z) * ht


def _tc_gru(a0, a1, h2, d0, d1, wz, wr, wh, lz, lr, lh, bz2, br2, bh2):
    row_spec = pl.BlockSpec((BR, F), lambda i: (i, 0))
    col_spec = pl.BlockSpec((BR, 1), lambda i: (i, 0))
    w_spec = pl.BlockSpec((F, F), lambda i: (0, 0))
    l_spec = pl.BlockSpec((2 * F, F), lambda i: (0, 0))
    b_spec = pl.BlockSpec((1, F), lambda i: (0, 0))
    return pl.pallas_call(
        _gru_body,
        grid=(N // BR,),
        in_specs=[row_spec, row_spec, row_spec, col_spec, col_spec,
                  w_spec, w_spec, w_spec, l_spec, l_spec, l_spec,
                  b_spec, b_spec, b_spec],
        out_specs=row_spec,
        out_shape=jax.ShapeDtypeStruct((N, F), jnp.float32),
    )(a0, a1, h2, d0, d1, wz, wr, wh, lz, lr, lh, bz2, br2, bh2)


def kernel(X, edge_index, edge_weight, H, Wz, bz, Wr, br, Wh, bh,
           LzW, Lzb, LrW, Lrb, LhW, Lhb):
    x2 = X.reshape(N, F)
    h2 = H.reshape(N, F)
    ei = edge_index.astype(jnp.int32)
    pad = EP - E
    row3 = jnp.pad(ei[0], (0, pad)).reshape(NW, NCHUNK, CHUNK)
    col3 = jnp.pad(ei[1], (0, pad)).reshape(NW, NCHUNK, CHUNK)
    ew3 = jnp.pad(edge_weight, (0, pad)).reshape(NW, NCHUNK, CHUNK)
    zeros_n = jnp.zeros((NPAD,), jnp.float32)
    zeros_nf = jnp.zeros((NPAD, F), jnp.float32)

    degp = _sc_deg(col3, ew3, zeros_n)
    d0 = degp[0].reshape(NPAD, 1)
    d1 = degp[1].reshape(NPAD, 1)
    x2p = jnp.pad(x2, ((0, NPAD - N), (0, 0)))
    xs = _tc_xs(x2p, d0, d1)
    aggp = _sc_agg(row3, col3, ew3, xs, zeros_nf)

    bz2 = (bz @ LzW[:F] + Lzb).reshape(1, F)
    br2 = (br @ LrW[:F] + Lrb).reshape(1, F)
    bh2 = (bh @ LhW[:F] + Lhb).reshape(1, F)

    out = _tc_gru(aggp[0], aggp[1], h2, d0, d1,
                  Wz, Wr, Wh, LzW, LrW, LhW, bz2, br2, bh2)
    return out.reshape(1, N, F)


# speedup vs baseline: 1.0004x; 1.0004x over previous
import functools

import jax
import jax.numpy as jnp
from jax import lax
from jax.experimental import pallas as pl
from jax.experimental.pallas import tpu as pltpu
from jax.experimental.pallas import tpu_sc as plsc

N = 10000
E = 320000
F = 128
NC = 2
NS = 16
NW = NC * NS
CHUNK = 96
NCHUNK = 108
EP = NW * NCHUNK * CHUNK
NPAD = 10240
BR = 1000
BRP = 1024


def _sc_deg(col3, ew3, zeros_n):
    mesh = plsc.VectorSubcoreMesh(core_axis_name="c", subcore_axis_name="s")

    @functools.partial(
        pl.kernel,
        mesh=mesh,
        out_type=jax.ShapeDtypeStruct((NC, NPAD), jnp.float32),
        scratch_types=[
            pltpu.VMEM((NCHUNK, CHUNK), jnp.int32),
            pltpu.VMEM((NCHUNK, CHUNK), jnp.float32),
            pltpu.VMEM_SHARED((NPAD,), jnp.float32),
            pltpu.SemaphoreType.DMA,
        ],
    )
    def k(colh, ewh, zh, degout, colv, eww, deg, sem):
        c = lax.axis_index("c")
        s = lax.axis_index("s")
        w = c * NS + s
        sl = pl.ds(s * (NPAD // NS), NPAD // NS)
        pltpu.sync_copy(zh.at[sl], deg.at[sl])
        plsc.subcore_barrier()
        pltpu.sync_copy(colh.at[w], colv)
        pltpu.sync_copy(ewh.at[w], eww)

        def group(g, carry):
            base = g * 6
            for t in range(6):
                pltpu.async_copy(eww.at[base + t],
                                 deg.at[colv.at[base + t]], sem, add=True)
            for t in range(6):
                pltpu.make_async_copy(eww.at[base + t],
                                      deg.at[colv.at[base + t]], sem).wait()
            return carry

        lax.fori_loop(0, NCHUNK // 6, group, 0)
        plsc.subcore_barrier()
        pltpu.sync_copy(deg.at[sl], degout.at[c, sl])

    return k(col3, ew3, zeros_n)


def _xs_body(x_ref, d0_ref, d1_ref, o_ref):
    ds = lax.rsqrt(d0_ref[...] + d1_ref[...] + 1.0)
    o_ref[...] = x_ref[...] * ds


def _tc_xs(x2, d0, d1):
    return pl.pallas_call(
        _xs_body,
        grid=(NPAD // BRP,),
        in_specs=[
            pl.BlockSpec((BRP, F), lambda i: (i, 0)),
            pl.BlockSpec((BRP, 1), lambda i: (i, 0)),
            pl.BlockSpec((BRP, 1), lambda i: (i, 0)),
        ],
        out_specs=pl.BlockSpec((BRP, F), lambda i: (i, 0)),
        out_shape=jax.ShapeDtypeStruct((NPAD, F), jnp.float32),
    )(x2, d0, d1)


def _sc_agg(row3, col3, ew3, xs, zeros_nf):
    mesh = plsc.VectorSubcoreMesh(core_axis_name="c", subcore_axis_name="s")

    @functools.partial(
        pl.kernel,
        mesh=mesh,
        out_type=jax.ShapeDtypeStruct((NC, NPAD, F), jnp.float32),
        scratch_types=[
            pltpu.VMEM((6, CHUNK), jnp.int32),
            pltpu.VMEM((6, CHUNK), jnp.int32),
            pltpu.VMEM((6, CHUNK), jnp.float32),
            pltpu.VMEM((CHUNK, F), jnp.float32),
            pltpu.VMEM((CHUNK, F), jnp.float32),
            pltpu.VMEM((CHUNK, F), jnp.float32),
            pltpu.SemaphoreType.DMA,
            pltpu.SemaphoreType.DMA,
            pltpu.SemaphoreType.DMA,
            pltpu.SemaphoreType.DMA,
            pltpu.SemaphoreType.DMA,
            pltpu.SemaphoreType.DMA,
            pltpu.SemaphoreType.DMA,
            pltpu.SemaphoreType.DMA,
            pltpu.SemaphoreType.DMA,
            pltpu.SemaphoreType.DMA,
            pltpu.SemaphoreType.DMA,
            pltpu.SemaphoreType.DMA,
            pltpu.VMEM_SHARED((NPAD, F), jnp.float32),
        ],
    )
    def k(rowh, colh, ewh, xsh, zh, aggout, rowi, coli, ewc,
          pb0, pb1, pb2,
          is0, is1, is2, is3, is4, is5, gs0, gs1, gs2, ss0, ss1, ss2, agg):
        c = lax.axis_index("c")
        s = lax.axis_index("s")
        w = c * NS + s
        sl = pl.ds(s * (NPAD // NS), NPAD // NS)

        @pl.when(c == 0)
        def _():
            pltpu.sync_copy(xsh.at[sl], agg.at[sl])

        @pl.when(c != 0)
        def _():
            pltpu.sync_copy(zh.at[sl], agg.at[sl])

        plsc.subcore_barrier()

        pbufs = (pb0, pb1, pb2)
        isems = (is0, is1, is2, is3, is4, is5)
        gsems = (gs0, gs1, gs2)
        ssems = (ss0, ss1, ss2)
        dnums = lax.GatherDimensionNumbers(
            offset_dims=(), collapsed_slice_dims=(0,), start_index_map=(0,))

        def idx_start(cix, q):
            pltpu.async_copy(rowh.at[w, cix], rowi.at[q], isems[q])
            pltpu.async_copy(colh.at[w, cix], coli.at[q], isems[q])
            pltpu.async_copy(ewh.at[w, cix], ewc.at[q], isems[q])

        def idx_wait(cix, q):
            pltpu.make_async_copy(rowh.at[w, cix], rowi.at[q], isems[q]).wait()
            pltpu.make_async_copy(colh.at[w, cix], coli.at[q], isems[q]).wait()
            pltpu.make_async_copy(ewh.at[w, cix], ewc.at[q], isems[q]).wait()

        def gather_start(q, b):
            pltpu.async_copy(xsh.at[rowi.at[q]], pbufs[b], gsems[b])

        def gather_wait(q, b):
            pltpu.make_async_copy(xsh.at[rowi.at[q]], pbufs[b],
                                  gsems[b]).wait()

        def scatter_start(q, b):
            pltpu.async_copy(pbufs[b], agg.at[coli.at[q]], ssems[b],
                             add=True)

        def scatter_wait(q, b):
            pltpu.make_async_copy(pbufs[b], agg.at[coli.at[q]],
                                  ssems[b]).wait()

        def scale(q, b):
            buf = pbufs[b]

            def grp(g, c2):
                wgrp = ewc[q, pl.ds(g * 16, 16)]
                for l in range(16):
                    wv = lax.gather(
                        wgrp, jnp.full((16, 1), l, jnp.int32), dnums, (1,),
                        mode=lax.GatherScatterMode.PROMISE_IN_BOUNDS)
                    kk = g * 16 + l
                    for t in range(F // 16):
                        buf[kk, pl.ds(t * 16, 16)] = (
                            buf[kk, pl.ds(t * 16, 16)] * wv)
                return c2

            lax.fori_loop(0, CHUNK // 16, grp, 0)

        for c0 in range(4):
            idx_start(c0, c0)
        idx_wait(0, 0)
        gather_start(0, 0)
        for c0 in (0, 1):
            gather_wait(c0 % 6, c0 % 3)
            idx_wait(c0 + 1, c0 + 1)
            gather_start((c0 + 1) % 6, (c0 + 1) % 3)
            idx_start(c0 + 4, c0 + 4)
            scale(c0 % 6, c0 % 3)
            scatter_start(c0 % 6, c0 % 3)

        def steady(i, carry):
            j = 2 + i * 6
            for p in range(6):
                cix = j + p
                q = (2 + p) % 6
                b = (2 + p) % 3
                qn = (3 + p) % 6
                bn = (p + 0) % 3
                gather_wait(q, b)
                scatter_wait((p + 0) % 6, bn)
                idx_wait(cix + 1, qn)
                gather_start(qn, bn)
                idx_start(cix + 4, (p + 0) % 6)
                scale(q, b)
                scatter_start(q, b)
            return carry

        lax.fori_loop(0, (NCHUNK - 6) // 6, steady, 0)

        for c0 in (NCHUNK - 4, NCHUNK - 3, NCHUNK - 2):
            gather_wait(c0 % 6, c0 % 3)
            scatter_wait((c0 - 2) % 6, (c0 + 1) % 3)
            idx_wait(c0 + 1, (c0 + 1) % 6)
            gather_start((c0 + 1) % 6, (c0 + 1) % 3)
            scale(c0 % 6, c0 % 3)
            scatter_start(c0 % 6, c0 % 3)
        c0 = NCHUNK - 1
        gather_wait(c0 % 6, c0 % 3)
        scatter_wait((c0 - 2) % 6, (c0 + 1) % 3)
        scale(c0 % 6, c0 % 3)
        scatter_start(c0 % 6, c0 % 3)
        scatter_wait((NCHUNK - 2) % 6, (NCHUNK - 2) % 3)
        scatter_wait((NCHUNK - 1) % 6, (NCHUNK - 1) % 3)

        plsc.subcore_barrier()
        pltpu.sync_copy(agg.at[sl], aggout.at[c, sl])

    return k(row3, col3, ew3, xs, zeros_nf)


def _gru_body(a0_ref, a1_ref, h_ref, d0_ref, d1_ref,
              wz_ref, wr_ref, wh_ref, lz_ref, lr_ref, lh_ref,
              bz_ref, br_ref, bh_ref, o_ref):
    f32 = jnp.float32
    ds = lax.rsqrt(d0_ref[...] + d1_ref[...] + 1.0)
    agg = (a0_ref[...] + a1_ref[...]) * ds
    h = h_ref[...]

    def gate(w_ref, l_ref, b_ref, hv):
        lt = l_ref[0:F, :]
        lb = l_ref[F:2 * F, :]
        a = jnp.dot(w_ref[...], lt, preferred_element_type=f32)
        pre = (jnp.dot(agg, a, preferred_element_type=f32)
               + jnp.dot(hv, lb, preferred_element_type=f32)
               + b_ref[...])
        return pre

    z = jax.nn.sigmoid(gate(wz_ref, lz_ref, bz_ref, h))
    r = jax.nn.sigmoid(gate(wr_ref, lr_ref, br_ref, h))
    ht = jnp.tanh(gate(wh_ref, lh_ref, bh_ref, h * r))
    o_ref[...] = z * h + (1.0 - z) * ht


def _tc_gru(a0, a1, h2, d0, d1, wz, wr, wh, lz, lr, lh, bz2, br2, bh2):
    row_spec = pl.BlockSpec((BR, F), lambda i: (i, 0))
    col_spec = pl.BlockSpec((BR, 1), lambda i: (i, 0))
    w_spec = pl.BlockSpec((F, F), lambda i: (0, 0))
    l_spec = pl.BlockSpec((2 * F, F), lambda i: (0, 0))
    b_spec = pl.BlockSpec((1, F), lambda i: (0, 0))
    return pl.pallas_call(
        _gru_body,
        grid=(N // BR,),
        in_specs=[row_spec, row_spec, row_spec, col_spec, col_spec,
                  w_spec, w_spec, w_spec, l_spec, l_spec, l_spec,
                  b_spec, b_spec, b_spec],
        out_specs=row_spec,
        out_shape=jax.ShapeDtypeStruct((N, F), jnp.float32),
    )(a0, a1, h2, d0, d1, wz, wr, wh, lz, lr, lh, bz2, br2, bh2)


def kernel(X, edge_index, edge_weight, H, Wz, bz, Wr, br, Wh, bh,
           LzW, Lzb, LrW, Lrb, LhW, Lhb):
    x2 = X.reshape(N, F)
    h2 = H.reshape(N, F)
    ei = edge_index.astype(jnp.int32)
    pad = EP - E
    row3 = jnp.pad(ei[0], (0, pad)).reshape(NW, NCHUNK, CHUNK)
    col3 = jnp.pad(ei[1], (0, pad)).reshape(NW, NCHUNK, CHUNK)
    ew3 = jnp.pad(edge_weight, (0, pad)).reshape(NW, NCHUNK, CHUNK)
    zeros_n = jnp.zeros((NPAD,), jnp.float32)
    zeros_nf = jnp.zeros((NPAD, F), jnp.float32)

    degp = _sc_deg(col3, ew3, zeros_n)
    d0 = degp[0].reshape(NPAD, 1)
    d1 = degp[1].reshape(NPAD, 1)
    x2p = jnp.pad(x2, ((0, NPAD - N), (0, 0)))
    xs = _tc_xs(x2p, d0, d1)
    aggp = _sc_agg(row3, col3, ew3, xs, zeros_nf)

    bz2 = (bz @ LzW[:F] + Lzb).reshape(1, F)
    br2 = (br @ LrW[:F] + Lrb).reshape(1, F)
    bh2 = (bh @ LhW[:F] + Lhb).reshape(1, F)

    out = _tc_gru(aggp[0], aggp[1], h2, d0, d1,
                  Wz, Wr, Wh, LzW, LrW, LhW, bz2, br2, bh2)
    return out.reshape(1, N, F)


# speedup vs baseline: 1.1666x; 1.1661x over previous
import functools

import jax
import jax.numpy as jnp
from jax import lax
from jax.experimental import pallas as pl
from jax.experimental.pallas import tpu as pltpu
from jax.experimental.pallas import tpu_sc as plsc

N = 10000
E = 320000
F = 128
NC = 2
NS = 16
NW = NC * NS
CHUNK = 128
NCHUNK = 80
EP = NW * NCHUNK * CHUNK
NPAD = 10240
BR = 1000
BRP = 1024


def _sc_deg(col3, ew3, zeros_n):
    mesh = plsc.VectorSubcoreMesh(core_axis_name="c", subcore_axis_name="s")

    @functools.partial(
        pl.kernel,
        mesh=mesh,
        out_type=jax.ShapeDtypeStruct((NC, NPAD), jnp.float32),
        scratch_types=[
            pltpu.VMEM((NCHUNK, CHUNK), jnp.int32),
            pltpu.VMEM((NCHUNK, CHUNK), jnp.float32),
            pltpu.VMEM_SHARED((NPAD,), jnp.float32),
            pltpu.SemaphoreType.DMA,
        ],
    )
    def k(colh, ewh, zh, degout, colv, eww, deg, sem):
        c = lax.axis_index("c")
        s = lax.axis_index("s")
        w = c * NS + s
        sl = pl.ds(s * (NPAD // NS), NPAD // NS)
        pltpu.sync_copy(zh.at[sl], deg.at[sl])
        plsc.subcore_barrier()
        pltpu.sync_copy(colh.at[w], colv)
        pltpu.sync_copy(ewh.at[w], eww)

        def group(g, carry):
            base = g * 4
            for t in range(4):
                pltpu.async_copy(eww.at[base + t],
                                 deg.at[colv.at[base + t]], sem, add=True)
            for t in range(4):
                pltpu.make_async_copy(eww.at[base + t],
                                      deg.at[colv.at[base + t]], sem).wait()
            return carry

        lax.fori_loop(0, NCHUNK // 4, group, 0)
        plsc.subcore_barrier()
        pltpu.sync_copy(deg.at[sl], degout.at[c, sl])

    return k(col3, ew3, zeros_n)


def _xs_body(x_ref, d0_ref, d1_ref, o_ref):
    ds = lax.rsqrt(d0_ref[...] + d1_ref[...] + 1.0)
    o_ref[...] = x_ref[...] * ds


def _tc_xs(x2, d0, d1):
    return pl.pallas_call(
        _xs_body,
        grid=(NPAD // BRP,),
        in_specs=[
            pl.BlockSpec((BRP, F), lambda i: (i, 0)),
            pl.BlockSpec((BRP, 1), lambda i: (i, 0)),
            pl.BlockSpec((BRP, 1), lambda i: (i, 0)),
        ],
        out_specs=pl.BlockSpec((BRP, F), lambda i: (i, 0)),
        out_shape=jax.ShapeDtypeStruct((NPAD, F), jnp.float32),
    )(x2, d0, d1)


def _sc_agg(row3, col3, ew3, xs, zeros_nf):
    mesh = plsc.VectorSubcoreMesh(core_axis_name="c", subcore_axis_name="s")

    @functools.partial(
        pl.kernel,
        mesh=mesh,
        out_type=jax.ShapeDtypeStruct((NC, NPAD, F), jnp.float32),
        scratch_types=[
            pltpu.VMEM((NCHUNK, CHUNK), jnp.int32),
            pltpu.VMEM((NCHUNK, CHUNK), jnp.int32),
            pltpu.VMEM((NCHUNK, CHUNK), jnp.float32),
            pltpu.VMEM((CHUNK, F), jnp.float32),
            pltpu.SemaphoreType.DMA,
            pltpu.SemaphoreType.DMA,
            pltpu.VMEM_SHARED((NPAD, F), jnp.float32),
        ],
    )
    def k(rowh, colh, ewh, xsh, zh, aggout, rowv, colv, eww,
          pb0, gs0, ss0, agg):
        c = lax.axis_index("c")
        s = lax.axis_index("s")
        w = c * NS + s
        sl = pl.ds(s * (NPAD // NS), NPAD // NS)

        @pl.when(c == 0)
        def _():
            pltpu.sync_copy(xsh.at[sl], agg.at[sl])

        @pl.when(c != 0)
        def _():
            pltpu.sync_copy(zh.at[sl], agg.at[sl])

        pltpu.sync_copy(rowh.at[w], rowv)
        pltpu.sync_copy(colh.at[w], colv)
        pltpu.sync_copy(ewh.at[w], eww)
        plsc.subcore_barrier()

        dnums = lax.GatherDimensionNumbers(
            offset_dims=(), collapsed_slice_dims=(0,), start_index_map=(0,))

        def scale(cix):
            def grp(g, c2):
                wgrp = eww[cix, pl.ds(g * 16, 16)]
                for l in range(16):
                    wv = lax.gather(
                        wgrp, jnp.full((16, 1), l, jnp.int32), dnums, (1,),
                        mode=lax.GatherScatterMode.PROMISE_IN_BOUNDS)
                    kk = g * 16 + l
                    for t in range(F // 16):
                        pb0[kk, pl.ds(t * 16, 16)] = (
                            pb0[kk, pl.ds(t * 16, 16)] * wv)
                return c2

            lax.fori_loop(0, CHUNK // 16, grp, 0)

        def body(cix, carry):
            pltpu.async_copy(xsh.at[rowv.at[cix]], pb0, gs0)
            pltpu.make_async_copy(xsh.at[rowv.at[cix]], pb0, gs0).wait()
            scale(cix)
            pltpu.async_copy(pb0, agg.at[colv.at[cix]], ss0, add=True)
            pltpu.make_async_copy(pb0, agg.at[colv.at[cix]], ss0).wait()
            return carry

        lax.fori_loop(0, NCHUNK, body, 0)

        plsc.subcore_barrier()
        pltpu.sync_copy(agg.at[sl], aggout.at[c, sl])

    return k(row3, col3, ew3, xs, zeros_nf)


def _gru_body(a0_ref, a1_ref, h_ref, d0_ref, d1_ref,
              wz_ref, wr_ref, wh_ref, lz_ref, lr_ref, lh_ref,
              bz_ref, br_ref, bh_ref, o_ref):
    f32 = jnp.float32
    ds = lax.rsqrt(d0_ref[...] + d1_ref[...] + 1.0)
    agg = (a0_ref[...] + a1_ref[...]) * ds
    h = h_ref[...]

    def gate(w_ref, l_ref, b_ref, hv):
        lt = l_ref[0:F, :]
        lb = l_ref[F:2 * F, :]
        a = jnp.dot(w_ref[...], lt, preferred_element_type=f32)
        pre = (jnp.dot(agg, a, preferred_element_type=f32)
               + jnp.dot(hv, lb, preferred_element_type=f32)
               + b_ref[...])
        return pre

    z = jax.nn.sigmoid(gate(wz_ref, lz_ref, bz_ref, h))
    r = jax.nn.sigmoid(gate(wr_ref, lr_ref, br_ref, h))
    ht = jnp.tanh(gate(wh_ref, lh_ref, bh_ref, h * r))
    o_ref[...] = z * h + (1.0 - z) * ht


def _tc_gru(a0, a1, h2, d0, d1, wz, wr, wh, lz, lr, lh, bz2, br2, bh2):
    row_spec = pl.BlockSpec((BR, F), lambda i: (i, 0))
    col_spec = pl.BlockSpec((BR, 1), lambda i: (i, 0))
    w_spec = pl.BlockSpec((F, F), lambda i: (0, 0))
    l_spec = pl.BlockSpec((2 * F, F), lambda i: (0, 0))
    b_spec = pl.BlockSpec((1, F), lambda i: (0, 0))
    return pl.pallas_call(
        _gru_body,
        grid=(N // BR,),
        in_specs=[row_spec, row_spec, row_spec, col_spec, col_spec,
                  w_spec, w_spec, w_spec, l_spec, l_spec, l_spec,
                  b_spec, b_spec, b_spec],
        out_specs=row_spec,
        out_shape=jax.ShapeDtypeStruct((N, F), jnp.float32),
    )(a0, a1, h2, d0, d1, wz, wr, wh, lz, lr, lh, bz2, br2, bh2)


def kernel(X, edge_index, edge_weight, H, Wz, bz, Wr, br, Wh, bh,
           LzW, Lzb, LrW, Lrb, LhW, Lhb):
    x2 = X.reshape(N, F)
    h2 = H.reshape(N, F)
    ei = edge_index.astype(jnp.int32)
    pad = EP - E
    row3 = jnp.pad(ei[0], (0, pad)).reshape(NW, NCHUNK, CHUNK)
    col3 = jnp.pad(ei[1], (0, pad)).reshape(NW, NCHUNK, CHUNK)
    ew3 = jnp.pad(edge_weight, (0, pad)).reshape(NW, NCHUNK, CHUNK)
    zeros_n = jnp.zeros((NPAD,), jnp.float32)
    zeros_nf = jnp.zeros((NPAD, F), jnp.float32)

    degp = _sc_deg(col3, ew3, zeros_n)
    d0 = degp[0].reshape(NPAD, 1)
    d1 = degp[1].reshape(NPAD, 1)
    x2p = jnp.pad(x2, ((0, NPAD - N), (0, 0)))
    xs = _tc_xs(x2p, d0, d1)
    aggp = _sc_agg(row3, col3, ew3, xs, zeros_nf)

    bz2 = (bz @ LzW[:F] + Lzb).reshape(1, F)
    br2 = (br @ LrW[:F] + Lrb).reshape(1, F)
    bh2 = (bh @ LhW[:F] + Lhb).reshape(1, F)

    out = _tc_gru(aggp[0], aggp[1], h2, d0, d1,
                  Wz, Wr, Wh, LzW, LrW, LhW, bz2, br2, bh2)
    return out.reshape(1, N, F)
